# baseline (speedup 1.0000x reference)
import functools

import jax
import jax.numpy as jnp
from jax import lax
from jax.experimental import pallas as pl
from jax.experimental.pallas import tpu as pltpu
from jax.experimental.pallas import tpu_sc as plsc

D = 128
_NC = 2
_NS = 16
_NW = _NC * _NS
_C = 40
_NBUF = 2


def _mlp1_body(x_ref, w1a_ref, w1b_ref, b1_ref, y1_ref, y2_ref):
    xb = x_ref[...]
    y1_ref[...] = (
        jnp.dot(xb, w1a_ref[...], preferred_element_type=jnp.float32)
        + b1_ref[...]
    )
    y2_ref[...] = jnp.dot(xb, w1b_ref[...], preferred_element_type=jnp.float32)


def _node_tables(x, W1, b1):
    n, d = x.shape
    bn = 1000
    y1, y2 = pl.pallas_call(
        _mlp1_body,
        grid=(n // bn,),
        in_specs=[
            pl.BlockSpec((bn, d), lambda i: (i, 0)),
            pl.BlockSpec((d, d), lambda i: (0, 0)),
            pl.BlockSpec((d, d), lambda i: (0, 0)),
            pl.BlockSpec((1, d), lambda i: (0, 0)),
        ],
        out_specs=[
            pl.BlockSpec((bn, d), lambda i: (i, 0)),
            pl.BlockSpec((bn, d), lambda i: (i, 0)),
        ],
        out_shape=[
            jax.ShapeDtypeStruct((n, d), jnp.float32),
            jax.ShapeDtypeStruct((n, d), jnp.float32),
        ],
    )(x, W1[:d], W1[d:], b1.reshape(1, d))
    return y1, y2


def _make_sc_kernel(E_pad, C, nbuf):
    epw = E_pad // _NW
    n_chunks = epw // C
    assert epw % C == 0 and n_chunks % nbuf == 0 and C % 8 == 0
    mesh = plsc.VectorSubcoreMesh(core_axis_name="c", subcore_axis_name="s")

    @functools.partial(
        pl.kernel,
        mesh=mesh,
        out_type=jax.ShapeDtypeStruct((E_pad * 16,), jnp.float32),
        scratch_types=[
            pltpu.VMEM((nbuf, C), jnp.int32),
            pltpu.VMEM((nbuf, C), jnp.int32),
            pltpu.VMEM((nbuf, C, D), jnp.float32),
            pltpu.VMEM((nbuf, C, D), jnp.float32),
            pltpu.VMEM((nbuf, C * 16), jnp.float32),
            pltpu.VMEM((D,), jnp.float32),
        ] + [pltpu.SemaphoreType.DMA] * (3 * nbuf),
    )
    def sc_edge_mlp(y1_hbm, y2_hbm, src_hbm, dst_hbm, w2_hbm, out_hbm,
                    src_v, dst_v, rows1, rows2, outc, w2_v, *sems):
        sem_i = sems[0:nbuf]
        sem_g = sems[nbuf:2 * nbuf]
        sem_o = sems[2 * nbuf:3 * nbuf]
        wid = lax.axis_index("s") * _NC + lax.axis_index("c")
        base = wid * epw
        pltpu.sync_copy(w2_hbm, w2_v)
        w2r = [w2_v[pl.ds(k * 16, 16)] for k in range(D // 16)]

        def issue_idx(c, b):
            off = base + c * C
            pltpu.async_copy(src_hbm.at[pl.ds(off, C)], src_v.at[b], sem_i[b])
            pltpu.async_copy(dst_hbm.at[pl.ds(off, C)], dst_v.at[b], sem_i[b])

        def wait_idx(b):
            pltpu.make_async_copy(src_hbm.at[pl.ds(0, C)], src_v.at[b],
                                  sem_i[b]).wait()
            pltpu.make_async_copy(dst_hbm.at[pl.ds(0, C)], dst_v.at[b],
                                  sem_i[b]).wait()

        def issue_gather(b):
            pltpu.async_copy(y1_hbm.at[src_v.at[b]], rows1.at[b], sem_g[b])
            pltpu.async_copy(y2_hbm.at[dst_v.at[b]], rows2.at[b], sem_g[b])

        def wait_gather(b):
            pltpu.make_async_copy(y1_hbm.at[src_v.at[b]], rows1.at[b],
                                  sem_g[b]).wait()
            pltpu.make_async_copy(y2_hbm.at[dst_v.at[b]], rows2.at[b],
                                  sem_g[b]).wait()

        def issue_store(c, b):
            off = (base + c * C) * 16
            pltpu.async_copy(outc.at[b], out_hbm.at[pl.ds(off, C * 16)],
                             sem_o[b])

        def wait_store(b):
            pltpu.make_async_copy(outc.at[b], out_hbm.at[pl.ds(0, C * 16)],
                                  sem_o[b]).wait()

        for c in range(nbuf):
            issue_idx(c, c)
        for c in range(nbuf - 1):
            wait_idx(c)
            issue_gather(c)

        def step(si, carry):
            for b in range(nbuf):
                s = nbuf * si + b
                wait_gather(b)

                @pl.when(s + nbuf < n_chunks)
                def _():
                    issue_idx(s + nbuf, b)

                pg = (b + nbuf - 1) % nbuf

                @pl.when(s + nbuf - 1 < n_chunks)
                def _():
                    wait_idx(pg)
                    issue_gather(pg)

                @pl.when(s >= nbuf)
                def _():
                    wait_store(b)

                def edge(e, c2):
                    acc0 = jnp.zeros((16,), jnp.float32)
                    acc1 = jnp.zeros((16,), jnp.float32)
                    for k in range(D // 16):
                        f = k * 16
                        av = rows1[b, e, pl.ds(f, 16)]
                        bv = rows2[b, e, pl.ds(f, 16)]
                        z = jnp.maximum(av + bv, 0.0)
                        if k % 2 == 0:
                            acc0 = acc0 + z * w2r[k]
                        else:
                            acc1 = acc1 + z * w2r[k]
                    outc[b, pl.ds(e * 16, 16)] = acc0 + acc1
                    return c2

                lax.fori_loop(0, C, edge, 0, unroll=4)
                issue_store(s, b)
            return carry

        lax.fori_loop(0, n_chunks // nbuf, step, 0)
        for b in range(nbuf):
            wait_store(b)

    return sc_edge_mlp


def _lane_sum_body(p_ref, b2_ref, o_ref):
    i = lax.broadcasted_iota(jnp.int32, (D, 8), 0)
    j = lax.broadcasted_iota(jnp.int32, (D, 8), 1)
    m = (i // 16 == j).astype(jnp.float32)
    o_ref[...] = (
        jnp.dot(p_ref[...], m, preferred_element_type=jnp.float32)
        + b2_ref[0]
    )


def _lane_sum(partials, E_pad, b2):
    rows = E_pad // 8
    br = 4000
    p2 = partials.reshape(rows, D)
    out = pl.pallas_call(
        _lane_sum_body,
        grid=(rows // br,),
        in_specs=[
            pl.BlockSpec((br, D), lambda i: (i, 0)),
            pl.BlockSpec(memory_space=pltpu.SMEM),
        ],
        out_specs=pl.BlockSpec((br, 8), lambda i: (i, 0)),
        out_shape=jax.ShapeDtypeStruct((rows, 8), jnp.float32),
    )(p2, b2)
    return out.reshape(E_pad)


def kernel(x, edge_index, W1, b1, W2, b2):
    y1, y2 = _node_tables(x, W1, b1)
    src = edge_index[0]
    dst = edge_index[1]
    E = src.shape[0]
    grain = _NW * _C * _NBUF
    E_pad = ((E + grain - 1) // grain) * grain
    if E_pad != E:
        pad = jnp.zeros((E_pad - E,), jnp.int32)
        src = jnp.concatenate([src, pad])
        dst = jnp.concatenate([dst, pad])
    sc = _make_sc_kernel(E_pad, _C, _NBUF)
    partials = sc(y1, y2, src, dst, W2.reshape(-1))
    scores = _lane_sum(partials, E_pad, b2)
    if E_pad != E:
        scores = scores[:E]
    return scores


# speedup vs baseline: 1.0286x; 1.0286x over previous
import functools

import jax
import jax.numpy as jnp
from jax import lax
from jax.experimental import pallas as pl
from jax.experimental.pallas import tpu as pltpu
from jax.experimental.pallas import tpu_sc as plsc

D = 128
_NC = 2
_NS = 16
_NW = _NC * _NS
_C = 40
_NBUF = 2


def _mlp1_body(x_ref, w1a_ref, w1b_ref, b1_ref, y1_ref, y2_ref):
    xb = x_ref[...]
    y1_ref[...] = (
        jnp.dot(xb, w1a_ref[...], preferred_element_type=jnp.float32)
        + b1_ref[...]
    )
    y2_ref[...] = jnp.dot(xb, w1b_ref[...], preferred_element_type=jnp.float32)


def _node_tables(x, W1, b1):
    n, d = x.shape
    bn = 1000
    y1, y2 = pl.pallas_call(
        _mlp1_body,
        grid=(n // bn,),
        in_specs=[
            pl.BlockSpec((bn, d), lambda i: (i, 0)),
            pl.BlockSpec((d, d), lambda i: (0, 0)),
            pl.BlockSpec((d, d), lambda i: (0, 0)),
            pl.BlockSpec((1, d), lambda i: (0, 0)),
        ],
        out_specs=[
            pl.BlockSpec((bn, d), lambda i: (i, 0)),
            pl.BlockSpec((bn, d), lambda i: (i, 0)),
        ],
        out_shape=[
            jax.ShapeDtypeStruct((n, d), jnp.float32),
            jax.ShapeDtypeStruct((n, d), jnp.float32),
        ],
    )(x, W1[:d], W1[d:], b1.reshape(1, d))
    return y1, y2


def _make_sc_kernel(E_pad, C, nbuf):
    epw = E_pad // _NW
    n_chunks = epw // C
    assert epw % C == 0 and n_chunks % nbuf == 0 and C % 8 == 0
    mesh = plsc.VectorSubcoreMesh(core_axis_name="c", subcore_axis_name="s")

    @functools.partial(
        pl.kernel,
        mesh=mesh,
        out_type=jax.ShapeDtypeStruct((E_pad * 16,), jnp.float32),
        scratch_types=[
            pltpu.VMEM((nbuf, C), jnp.int32),
            pltpu.VMEM((nbuf, C), jnp.int32),
            pltpu.VMEM((nbuf, C, D), jnp.float32),
            pltpu.VMEM((nbuf, C, D), jnp.float32),
            pltpu.VMEM((nbuf, C * 16), jnp.float32),
            pltpu.VMEM((D,), jnp.float32),
        ] + [pltpu.SemaphoreType.DMA] * (3 * nbuf),
    )
    def sc_edge_mlp(y1_hbm, y2_hbm, ei_hbm, w2_hbm, out_hbm,
                    src_v, dst_v, rows1, rows2, outc, w2_v, *sems):
        sem_i = sems[0:nbuf]
        sem_g = sems[nbuf:2 * nbuf]
        sem_o = sems[2 * nbuf:3 * nbuf]
        wid = lax.axis_index("s") * _NC + lax.axis_index("c")
        base = wid * epw
        pltpu.sync_copy(w2_hbm, w2_v)
        w2r = [w2_v[pl.ds(k * 16, 16)] for k in range(D // 16)]

        def issue_idx(c, b):
            off = base + c * C
            pltpu.async_copy(ei_hbm.at[pl.ds(off, C)], src_v.at[b], sem_i[b])
            pltpu.async_copy(ei_hbm.at[pl.ds(E_pad + off, C)], dst_v.at[b],
                             sem_i[b])

        def wait_idx(b):
            pltpu.make_async_copy(ei_hbm.at[pl.ds(0, C)], src_v.at[b],
                                  sem_i[b]).wait()
            pltpu.make_async_copy(ei_hbm.at[pl.ds(0, C)], dst_v.at[b],
                                  sem_i[b]).wait()

        def issue_gather(b):
            pltpu.async_copy(y1_hbm.at[src_v.at[b]], rows1.at[b], sem_g[b])
            pltpu.async_copy(y2_hbm.at[dst_v.at[b]], rows2.at[b], sem_g[b])

        def wait_gather(b):
            pltpu.make_async_copy(y1_hbm.at[src_v.at[b]], rows1.at[b],
                                  sem_g[b]).wait()
            pltpu.make_async_copy(y2_hbm.at[dst_v.at[b]], rows2.at[b],
                                  sem_g[b]).wait()

        def issue_store(c, b):
            off = (base + c * C) * 16
            pltpu.async_copy(outc.at[b], out_hbm.at[pl.ds(off, C * 16)],
                             sem_o[b])

        def wait_store(b):
            pltpu.make_async_copy(outc.at[b], out_hbm.at[pl.ds(0, C * 16)],
                                  sem_o[b]).wait()

        for c in range(nbuf):
            issue_idx(c, c)
        for c in range(nbuf - 1):
            wait_idx(c)
            issue_gather(c)

        def step(si, carry):
            for b in range(nbuf):
                s = nbuf * si + b
                wait_gather(b)

                @pl.when(s + nbuf < n_chunks)
                def _():
                    issue_idx(s + nbuf, b)

                pg = (b + nbuf - 1) % nbuf

                @pl.when(s + nbuf - 1 < n_chunks)
                def _():
                    wait_idx(pg)
                    issue_gather(pg)

                @pl.when(s >= nbuf)
                def _():
                    wait_store(b)

                def edge(e, c2):
                    acc0 = jnp.zeros((16,), jnp.float32)
                    acc1 = jnp.zeros((16,), jnp.float32)
                    for k in range(D // 16):
                        f = k * 16
                        av = rows1[b, e, pl.ds(f, 16)]
                        bv = rows2[b, e, pl.ds(f, 16)]
                        z = jnp.maximum(av + bv, 0.0)
                        if k % 2 == 0:
                            acc0 = acc0 + z * w2r[k]
                        else:
                            acc1 = acc1 + z * w2r[k]
                    outc[b, pl.ds(e * 16, 16)] = acc0 + acc1
                    return c2

                lax.fori_loop(0, C, edge, 0, unroll=4)
                issue_store(s, b)
            return carry

        lax.fori_loop(0, n_chunks // nbuf, step, 0)
        for b in range(nbuf):
            wait_store(b)

    return sc_edge_mlp


def _lane_sum_body(p_ref, b2_ref, o_ref):
    i = lax.broadcasted_iota(jnp.int32, (D, 8), 0)
    j = lax.broadcasted_iota(jnp.int32, (D, 8), 1)
    m = (i // 16 == j).astype(jnp.float32)
    o_ref[...] = (
        jnp.dot(p_ref[...], m, preferred_element_type=jnp.float32)
        + b2_ref[0]
    )


def _lane_sum(partials, E_pad, b2):
    rows = E_pad // 8
    br = 4000
    p2 = partials.reshape(rows, D)
    out = pl.pallas_call(
        _lane_sum_body,
        grid=(rows // br,),
        in_specs=[
            pl.BlockSpec((br, D), lambda i: (i, 0)),
            pl.BlockSpec(memory_space=pltpu.SMEM),
        ],
        out_specs=pl.BlockSpec((br, 8), lambda i: (i, 0)),
        out_shape=jax.ShapeDtypeStruct((rows, 8), jnp.float32),
    )(p2, b2)
    return out.reshape(E_pad)


def kernel(x, edge_index, W1, b1, W2, b2):
    y1, y2 = _node_tables(x, W1, b1)
    E = edge_index.shape[1]
    grain = _NW * _C * _NBUF
    E_pad = ((E + grain - 1) // grain) * grain
    if E_pad != E:
        ei = jnp.concatenate(
            [edge_index,
             jnp.zeros((2, E_pad - E), jnp.int32)], axis=1).reshape(-1)
    else:
        ei = edge_index.reshape(-1)
    sc = _make_sc_kernel(E_pad, _C, _NBUF)
    partials = sc(y1, y2, ei, W2.reshape(-1))
    scores = _lane_sum(partials, E_pad, b2)
    if E_pad != E:
        scores = scores[:E]
    return scores
